# baseline (speedup 1.0000x reference)
import functools

import jax
import jax.numpy as jnp
from jax import lax
from jax.experimental import pallas as pl
from jax.experimental.pallas import tpu as pltpu
from jax.experimental.pallas import tpu_sc as plsc

B, L, D = 4096, 50, 128
CAT_V, COL_V, STY_V = 100000, 1000, 1000
T = B * L

ROWS_PER_BLK = 4000
N_CAT_BLKS = CAT_V // ROWS_PER_BLK
FUSED_ROWS = (N_CAT_BLKS + 1) * ROWS_PER_BLK
COL_BASE = CAT_V
STY_BASE = CAT_V + COL_V

NC, NS = 2, 16
NW = NC * NS
BATCH_PER_W = B // NW
CHUNK = 64
N_CHUNKS = L * BATCH_PER_W // CHUNK
N_SETS = 4


def _build_body(cat_ref, col_ref, sty_ref, w_ref, b_ref, out_ref):
    i = pl.program_id(0)

    @pl.when(i < N_CAT_BLKS)
    def _():
        out_ref[...] = jnp.dot(cat_ref[...], w_ref[0:D, :],
                               preferred_element_type=jnp.float32)

    @pl.when(i == N_CAT_BLKS)
    def _():
        out_ref[0:COL_V, :] = (
            jnp.dot(col_ref[...], w_ref[D:2 * D, :],
                    preferred_element_type=jnp.float32) + b_ref[...])
        out_ref[COL_V:COL_V + STY_V, :] = jnp.dot(
            sty_ref[...], w_ref[2 * D:3 * D, :],
            preferred_element_type=jnp.float32)
        out_ref[COL_V + STY_V:ROWS_PER_BLK, :] = jnp.zeros(
            (ROWS_PER_BLK - COL_V - STY_V, D), jnp.float32)


def _build_fused_table(cat_table, color_table, style_table, W, b2d):
    return pl.pallas_call(
        _build_body,
        grid=(N_CAT_BLKS + 1,),
        in_specs=[
            pl.BlockSpec((ROWS_PER_BLK, D),
                         lambda i: (jnp.minimum(i, N_CAT_BLKS - 1), 0)),
            pl.BlockSpec((COL_V, D), lambda i: (0, 0)),
            pl.BlockSpec((STY_V, D), lambda i: (0, 0)),
            pl.BlockSpec((3 * D, D), lambda i: (0, 0)),
            pl.BlockSpec((1, D), lambda i: (0, 0)),
        ],
        out_specs=pl.BlockSpec((ROWS_PER_BLK, D), lambda i: (i, 0)),
        out_shape=jax.ShapeDtypeStruct((FUSED_ROWS, D), jnp.float32),
    )(cat_table, color_table, style_table, W, b2d)


def _sc_body(ftab, cid, colid, styid, out,
             idx0, idx1, idx2, idx3, buf0, buf1, buf2, buf3,
             isem0, isem1, isem2, isem3,
             gsem0, gsem1, gsem2, gsem3,
             osem0, osem1, osem2, osem3):
    wid = lax.axis_index("s") * NC + lax.axis_index("c")
    wb = wid * BATCH_PER_W

    idxs = (idx0, idx1, idx2, idx3)
    bufs = (buf0, buf1, buf2, buf3)
    isems = (isem0, isem1, isem2, isem3)
    gsems = (gsem0, gsem1, gsem2, gsem3)
    osems = (osem0, osem1, osem2, osem3)

    def coords(g):
        return g // 2, wb + (g % 2) * CHUNK

    def idx_copies(g, k):
        l, off = coords(g)
        return (
            pltpu.make_async_copy(cid.at[l, pl.ds(off, CHUNK)],
                                  idxs[k].at[0], isems[k]),
            pltpu.make_async_copy(colid.at[l, pl.ds(off, CHUNK)],
                                  idxs[k].at[1], isems[k]),
            pltpu.make_async_copy(styid.at[l, pl.ds(off, CHUNK)],
                                  idxs[k].at[2], isems[k]),
        )

    def gather_copies(k):
        return tuple(
            pltpu.make_async_copy(ftab.at[idxs[k].at[j]],
                                  bufs[k].at[pl.ds(j * CHUNK, CHUNK)],
                                  gsems[k])
            for j in range(3))

    def fire_idx(g, k):
        for c in idx_copies(g, k):
            c.start()

    def fire_gathers(g, k):
        for c in idx_copies(g, k):
            c.wait()
        for c in gather_copies(k):
            c.start()

    def wait_gathers(k):
        for c in gather_copies(k):
            c.wait()

    def compute(k):
        buf = bufs[k]

        def tok(t, c2):
            for c in range(D // 16):
                sl = pl.ds(c * 16, 16)
                v = buf[t, sl] + buf[CHUNK + t, sl] + buf[2 * CHUNK + t, sl]
                buf[t, sl] = jnp.maximum(v, 0.0)
            return c2
        lax.fori_loop(0, CHUNK, tok, 0)

    def put_copy(g, k):
        l, off = coords(g)
        return pltpu.make_async_copy(bufs[k].at[pl.ds(0, CHUNK)],
                                     out.at[l, pl.ds(off, CHUNK)], osems[k])

    def step(i, carry):
        for k in range(N_SETS):
            g = N_SETS * i + k
            wait_gathers(k)

            @pl.when(g + N_SETS < N_CHUNKS)
            def _(g=g, k=k):
                fire_idx(g + N_SETS, k)
            compute(k)
            put_copy(g, k).start()
            put_copy(g, k).wait()

            @pl.when(g + N_SETS < N_CHUNKS)
            def _(g=g, k=k):
                fire_gathers(g + N_SETS, k)
        return carry

    for k in range(N_SETS):
        fire_idx(k, k)
        fire_gathers(k, k)
    lax.fori_loop(0, N_CHUNKS // N_SETS, step, 0)


def _sc_fuse(ftab, cid, colid, styid):
    mesh = plsc.VectorSubcoreMesh(core_axis_name="c", subcore_axis_name="s")
    fn = functools.partial(
        pl.kernel,
        mesh=mesh,
        out_type=jax.ShapeDtypeStruct((L, B, D), jnp.float32),
        scratch_types=(
            [pltpu.VMEM((3, CHUNK), jnp.int32)] * N_SETS
            + [pltpu.VMEM((3 * CHUNK, D), jnp.float32)] * N_SETS
            + [pltpu.SemaphoreType.DMA] * (3 * N_SETS)
        ),
    )(_sc_body)
    return fn(ftab, cid, colid, styid)


def kernel(category_ids, color_ids, style_ids, cat_table, color_table,
           style_table, W, b):
    ftab = _build_fused_table(cat_table, color_table, style_table, W,
                              b.reshape(1, D))
    out_lbd = _sc_fuse(ftab,
                       jnp.transpose(category_ids).astype(jnp.int32),
                       jnp.transpose(color_ids).astype(jnp.int32) + COL_BASE,
                       jnp.transpose(style_ids).astype(jnp.int32) + STY_BASE)
    return jnp.transpose(out_lbd, (1, 0, 2))
